# baseline (speedup 1.0000x reference)
import functools

import jax
import jax.numpy as jnp
from jax import lax
from jax.experimental import pallas as pl
from jax.experimental.pallas import tpu as pltpu
from jax.experimental.pallas import tpu_sc as plsc

BH = 16
BW = 16
HH = BH // 2


def _make_sc_gather(nB, C, rows_total):
    info = plsc.get_sparse_core_info()
    NC, NS = info.num_cores, info.num_subcores
    NW = NC * NS
    nblk = nB // NW
    half_rows = HH * C
    half_elems = HH * BW * C
    out_elems = BH * BW * C
    row_elems = BW * C
    span = (BW - 1) * C + 1

    mesh = plsc.VectorSubcoreMesh(core_axis_name="c", subcore_axis_name="s")

    @functools.partial(
        pl.kernel,
        mesh=mesh,
        compiler_params=pltpu.CompilerParams(
            needs_layout_passes=False, use_tc_tiling_on_sc=False
        ),
        out_type=jax.ShapeDtypeStruct((nB, 2, HH * BW, C), jnp.float32),
        scratch_types=[
            pltpu.VMEM((2, half_rows), jnp.int32),
            pltpu.VMEM((2, half_rows), jnp.int32),
            pltpu.VMEM((half_rows, BW), jnp.float32),
            pltpu.VMEM((half_rows, BW), jnp.float32),
            pltpu.VMEM((HH * BW, 129), jnp.float32),
            pltpu.VMEM((HH * BW, 129), jnp.float32),
            pltpu.SemaphoreType.DMA,
            pltpu.SemaphoreType.DMA,
            pltpu.SemaphoreType.DMA,
            pltpu.SemaphoreType.DMA,
            pltpu.SemaphoreType.DMA,
            pltpu.SemaphoreType.DMA,
        ],
    )
    def k(table_hbm, idx_hbm, out_hbm, idxA, idxB, bufA, bufB, outA, outB,
          semA, semB, wsemA, wsemB, isemA, isemB):
        wid = lax.axis_index("s") * NC + lax.axis_index("c")
        i0 = wid * nblk
        last = i0 + nblk - 1
        iota16 = lax.iota(jnp.int32, 16)

        def fire(idx_ref, h, buf_ref, sem):
            pltpu.async_copy(table_hbm.at[idx_ref.at[h]], buf_ref, sem)

        def drain_gather(idx_ref, buf_ref, sem):
            pltpu.make_async_copy(
                table_hbm.at[idx_ref.at[0]], buf_ref, sem
            ).wait()

        zero16 = jnp.zeros((16,), jnp.int32)
        one16 = jnp.ones((16,), jnp.int32)

        def transpose(buf_ref, out_ref):
            def a_body(a, carry):
                rowv = iota16 + a * BW

                @plsc.parallel_loop(0, C, unroll=16, carry=zero16)
                def tr(c, colv):
                    v = buf_ref[c * HH + a, :]
                    plsc.store_scatter(out_ref, [rowv, colv], v)
                    return colv + one16

                return carry

            lax.fori_loop(0, HH, a_body, 0)

        def issue_write(out_ref, i, h, wsem):
            pltpu.async_copy(
                out_ref.at[:, pl.ds(0, C)], out_hbm.at[i, h], wsem
            )

        def drain_write(out_ref, wsem):
            pltpu.make_async_copy(
                out_ref.at[:, pl.ds(0, C)], out_hbm.at[0, 0], wsem
            ).wait()

        def idx_fetch(i, idx_ref, isem):
            pltpu.async_copy(idx_hbm.at[i], idx_ref, isem)

        def idx_wait(idx_ref, isem):
            pltpu.make_async_copy(idx_hbm.at[0], idx_ref, isem).wait()

        def halfstep(idx_ref, buf_ref, sem, out_ref, wsem, i, h, t):
            drain_gather(idx_ref, buf_ref, sem)

            @pl.when(t > 0)
            def _():
                drain_write(out_ref, wsem)

            transpose(buf_ref, out_ref)
            issue_write(out_ref, i, h, wsem)

        pltpu.sync_copy(idx_hbm.at[i0], idxA)
        fire(idxA, 0, bufA, semA)
        idx_fetch(i0 + 1, idxB, isemB)

        def body(t, carry):
            b0 = i0 + 2 * t
            b1 = b0 + 1
            b2 = jnp.minimum(b1 + 1, last)
            b3 = jnp.minimum(b2 + 1, last)

            fire(idxA, 1, bufB, semB)
            halfstep(idxA, bufA, semA, outA, wsemA, b0, 0, t)
            idx_wait(idxB, isemB)
            fire(idxB, 0, bufA, semA)
            halfstep(idxA, bufB, semB, outB, wsemB, b0, 1, t)
            idx_fetch(b2, idxA, isemA)
            fire(idxB, 1, bufB, semB)
            halfstep(idxB, bufA, semA, outA, wsemA, b1, 0, t + 1)
            idx_wait(idxA, isemA)
            fire(idxA, 0, bufA, semA)
            halfstep(idxB, bufB, semB, outB, wsemB, b1, 1, t + 1)
            idx_fetch(b3, idxB, isemB)
            return carry

        lax.fori_loop(0, nblk // 2, body, 0)

        drain_gather(idxA, bufA, semA)
        idx_wait(idxB, isemB)
        drain_write(outA, wsemA)
        drain_write(outB, wsemB)

    return k


def kernel(x, indices, block_size, block_stride, block_offset):
    N, C, H, W = x.shape
    nB = indices.shape[0]
    wc = W // BW
    rows_total = N * C * H * wc

    n = indices[:, 0]
    ys = indices[:, 1] * block_stride[0] + block_offset[0]
    xs = indices[:, 2] * block_stride[1] + block_offset[1]

    h = ys[:, None] + jnp.arange(BH, dtype=jnp.int32)[None, :]
    tr = h >> 3
    hi = h & 7
    tc = (xs >> 7)[:, None]
    ck = ((xs & 127) >> 4)[:, None]
    prow16 = tr * ((W // 128) * 64) + hi * 8 + tc * 64 + ck
    plane = (n[:, None, None, None] * C
             + jnp.arange(C, dtype=jnp.int32)[None, None, :, None])
    idx_all = (plane * (H * wc)
               + prow16.reshape(nB, 2, 1, HH)).astype(jnp.int32)

    table = (x.reshape(N * C, H // 8, 8, W // 128, 128)
             .transpose(0, 1, 3, 2, 4)
             .reshape(rows_total, BW))
    idx_all = idx_all.reshape(nB, 2, HH * C)
    out = _make_sc_gather(nB, C, rows_total)(table, idx_all)
    return out.reshape(nB, BH, BW, C)


# speedup vs baseline: 1.2885x; 1.2885x over previous
import functools

import jax
import jax.numpy as jnp
from jax import lax
from jax.experimental import pallas as pl
from jax.experimental.pallas import tpu as pltpu
from jax.experimental.pallas import tpu_sc as plsc

BH = 16
BW = 16
HH = BH // 2


def _make_sc_gather(nB, C, rows_total):
    info = plsc.get_sparse_core_info()
    NC, NS = info.num_cores, info.num_subcores
    NW = NC * NS
    nblk = nB // NW
    half_rows = HH * C

    mesh = plsc.VectorSubcoreMesh(core_axis_name="c", subcore_axis_name="s")

    @functools.partial(
        pl.kernel,
        mesh=mesh,
        compiler_params=pltpu.CompilerParams(
            needs_layout_passes=False, use_tc_tiling_on_sc=False
        ),
        out_type=jax.ShapeDtypeStruct((nB, 2, HH * BW, C), jnp.float32),
        scratch_types=[
            pltpu.VMEM((2, 2, half_rows), jnp.int32),
            pltpu.VMEM((2, 2, half_rows), jnp.int32),
            pltpu.VMEM((half_rows, BW), jnp.float32),
            pltpu.VMEM((half_rows, BW), jnp.float32),
            pltpu.VMEM((half_rows, BW), jnp.float32),
            pltpu.VMEM((half_rows, BW), jnp.float32),
            pltpu.VMEM((HH * BW, 129), jnp.float32),
            pltpu.VMEM((HH * BW, 129), jnp.float32),
            pltpu.SemaphoreType.DMA,
            pltpu.SemaphoreType.DMA,
            pltpu.SemaphoreType.DMA,
            pltpu.SemaphoreType.DMA,
            pltpu.SemaphoreType.DMA,
            pltpu.SemaphoreType.DMA,
            pltpu.SemaphoreType.DMA,
            pltpu.SemaphoreType.DMA,
        ],
    )
    def k(table_hbm, idx_hbm, out_hbm, idxA, idxB, buf0, buf1, buf2, buf3,
          outA, outB, sem0, sem1, sem2, sem3, wsemA, wsemB, isemA, isemB):
        wid = lax.axis_index("s") * NC + lax.axis_index("c")
        i0 = wid * nblk
        p0 = i0 // 2
        lastp = p0 + nblk // 2 - 1
        iota16 = lax.iota(jnp.int32, 16)
        zero16 = jnp.zeros((16,), jnp.int32)
        one16 = jnp.ones((16,), jnp.int32)

        def fire(idx_ref, j, h, buf_ref, sem):
            pltpu.async_copy(table_hbm.at[idx_ref.at[j, h]], buf_ref, sem)

        def drain_gather(idx_ref, buf_ref, sem):
            pltpu.make_async_copy(
                table_hbm.at[idx_ref.at[0, 0]], buf_ref, sem
            ).wait()

        def transpose(buf_ref, out_ref):
            def a_body(a, carry):
                rbase = a * C
                rowv = iota16 + a * BW

                @plsc.parallel_loop(0, C, unroll=16, carry=zero16)
                def tr(c, colv):
                    v = buf_ref[rbase + c, :]
                    plsc.store_scatter(out_ref, [rowv, colv], v)
                    return colv + one16

                return carry

            lax.fori_loop(0, HH, a_body, 0)

        def issue_write(out_ref, i, h, wsem):
            pltpu.async_copy(
                out_ref.at[:, pl.ds(0, C)], out_hbm.at[i, h], wsem
            )

        def drain_write(out_ref, wsem):
            pltpu.make_async_copy(
                out_ref.at[:, pl.ds(0, C)], out_hbm.at[0, 0], wsem
            ).wait()

        def pair_fetch(p, idx_ref, isem):
            pltpu.async_copy(idx_hbm.at[p], idx_ref, isem)

        def pair_wait(idx_ref, isem):
            pltpu.make_async_copy(idx_hbm.at[0], idx_ref, isem).wait()

        def halfstep(idx_ref, buf_ref, sem, out_ref, wsem, i, h, t):
            drain_gather(idx_ref, buf_ref, sem)

            @pl.when(t > 0)
            def _():
                drain_write(out_ref, wsem)

            transpose(buf_ref, out_ref)
            issue_write(out_ref, i, h, wsem)

        pltpu.sync_copy(idx_hbm.at[p0], idxA)
        fire(idxA, 0, 0, buf0, sem0)
        fire(idxA, 0, 1, buf1, sem1)

        def body(t, carry):
            q0 = i0 + 4 * t
            q1 = q0 + 1
            q2 = q0 + 2
            q3 = q0 + 3
            pn1 = jnp.minimum(p0 + 2 * t + 1, lastp)
            pn2 = jnp.minimum(p0 + 2 * t + 2, lastp)

            pair_fetch(pn1, idxB, isemB)
            fire(idxA, 1, 0, buf2, sem2)
            halfstep(idxA, buf0, sem0, outA, wsemA, q0, 0, t)
            fire(idxA, 1, 1, buf3, sem3)
            halfstep(idxA, buf1, sem1, outB, wsemB, q0, 1, t)
            pair_wait(idxB, isemB)
            fire(idxB, 0, 0, buf0, sem0)
            halfstep(idxA, buf2, sem2, outA, wsemA, q1, 0, t + 1)
            fire(idxB, 0, 1, buf1, sem1)
            halfstep(idxA, buf3, sem3, outB, wsemB, q1, 1, t + 1)
            pair_fetch(pn2, idxA, isemA)
            fire(idxB, 1, 0, buf2, sem2)
            halfstep(idxB, buf0, sem0, outA, wsemA, q2, 0, t + 1)
            fire(idxB, 1, 1, buf3, sem3)
            halfstep(idxB, buf1, sem1, outB, wsemB, q2, 1, t + 1)
            pair_wait(idxA, isemA)
            fire(idxA, 0, 0, buf0, sem0)
            halfstep(idxB, buf2, sem2, outA, wsemA, q3, 0, t + 1)
            fire(idxA, 0, 1, buf1, sem1)
            halfstep(idxB, buf3, sem3, outB, wsemB, q3, 1, t + 1)
            return carry

        lax.fori_loop(0, nblk // 4, body, 0)

        drain_gather(idxA, buf0, sem0)
        drain_gather(idxA, buf1, sem1)
        drain_write(outA, wsemA)
        drain_write(outB, wsemB)

    return k


def kernel(x, indices, block_size, block_stride, block_offset):
    N, C, H, W = x.shape
    nB = indices.shape[0]
    wc = W // BW
    rows_total = N * C * H * wc

    n = indices[:, 0]
    ys = indices[:, 1] * block_stride[0] + block_offset[0]
    xs = indices[:, 2] * block_stride[1] + block_offset[1]

    h = ys[:, None] + jnp.arange(BH, dtype=jnp.int32)[None, :]
    tr = h >> 3
    hi = h & 7
    tc = (xs >> 7)[:, None]
    ck = ((xs & 127) >> 4)[:, None]
    prow16 = tr * ((W // 128) * 64) + hi * 8 + tc * 64 + ck
    plane = (n[:, None, None] * C
             + jnp.arange(C, dtype=jnp.int32)[None, None, :])
    idx_all = (plane * (H * wc)
               + prow16[:, :, None]).astype(jnp.int32)

    table = (x.reshape(N * C, H // 8, 8, W // 128, 128)
             .transpose(0, 1, 3, 2, 4)
             .reshape(rows_total, BW))
    idx_all = idx_all.reshape(nB // 2, 2, 2, HH * C)
    out = _make_sc_gather(nB, C, rows_total)(table, idx_all)
    return out.reshape(nB, BH, BW, C)


# speedup vs baseline: 1.3530x; 1.0501x over previous
import functools

import jax
import jax.numpy as jnp
from jax import lax
from jax.experimental import pallas as pl
from jax.experimental.pallas import tpu as pltpu
from jax.experimental.pallas import tpu_sc as plsc

BH = 16
BW = 16
HH = BH // 2


def _make_sc_gather(nB, C, rows_total, plane_rows):
    info = plsc.get_sparse_core_info()
    NC, NS = info.num_cores, info.num_subcores
    NW = NC * NS
    nblk = nB // NW
    half_rows = HH * C

    mesh = plsc.VectorSubcoreMesh(core_axis_name="c", subcore_axis_name="s")

    @functools.partial(
        pl.kernel,
        mesh=mesh,
        compiler_params=pltpu.CompilerParams(
            needs_layout_passes=False, use_tc_tiling_on_sc=False
        ),
        out_type=jax.ShapeDtypeStruct((nB, 2, HH * BW, C), jnp.float32),
        scratch_types=[
            pltpu.VMEM((nblk, BH), jnp.int32),
            pltpu.VMEM((4, half_rows), jnp.int32),
            pltpu.VMEM((half_rows, BW), jnp.float32),
            pltpu.VMEM((half_rows, BW), jnp.float32),
            pltpu.VMEM((half_rows, BW), jnp.float32),
            pltpu.VMEM((half_rows, BW), jnp.float32),
            pltpu.VMEM((HH * BW, 129), jnp.float32),
            pltpu.VMEM((HH * BW, 129), jnp.float32),
            pltpu.SemaphoreType.DMA,
            pltpu.SemaphoreType.DMA,
            pltpu.SemaphoreType.DMA,
            pltpu.SemaphoreType.DMA,
            pltpu.SemaphoreType.DMA,
            pltpu.SemaphoreType.DMA,
        ],
    )
    def k(table_hbm, pb_hbm, out_hbm, pb_v, idx_v, buf0, buf1, buf2, buf3,
          outA, outB, sem0, sem1, sem2, sem3, wsemA, wsemB):
        wid = lax.axis_index("s") * NC + lax.axis_index("c")
        i0 = wid * nblk
        iota16 = lax.iota(jnp.int32, 16)
        zero16 = jnp.zeros((16,), jnp.int32)
        one16 = jnp.ones((16,), jnp.int32)
        coffs = [(iota16 + cg * 16) * plane_rows for cg in range(C // 16)]

        def build(slot, blk, h):
            rv = pb_v[blk, :]
            for a in range(HH):
                s = rv[h * HH + a]
                for cg in range(C // 16):
                    idx_v[slot, pl.ds(a * C + cg * 16, 16)] = coffs[cg] + s

        def fire(slot, buf_ref, sem):
            pltpu.async_copy(table_hbm.at[idx_v.at[slot]], buf_ref, sem)

        def drain_gather(buf_ref, sem):
            pltpu.make_async_copy(
                table_hbm.at[idx_v.at[0]], buf_ref, sem
            ).wait()

        def transpose(buf_ref, out_ref):
            def a_body(a, carry):
                rbase = a * C
                rowv = iota16 + a * BW

                @plsc.parallel_loop(0, C, unroll=16, carry=zero16)
                def tr(c, colv):
                    v = buf_ref[rbase + c, :]
                    plsc.store_scatter(out_ref, [rowv, colv], v)
                    return colv + one16

                return carry

            lax.fori_loop(0, HH, a_body, 0)

        def issue_write(out_ref, i, h, wsem):
            pltpu.async_copy(
                out_ref.at[:, pl.ds(0, C)], out_hbm.at[i, h], wsem
            )

        def drain_write(out_ref, wsem):
            pltpu.make_async_copy(
                out_ref.at[:, pl.ds(0, C)], out_hbm.at[0, 0], wsem
            ).wait()

        def halfstep(buf_ref, sem, out_ref, wsem, i, h, t):
            drain_gather(buf_ref, sem)

            @pl.when(t > 0)
            def _():
                drain_write(out_ref, wsem)

            transpose(buf_ref, out_ref)
            issue_write(out_ref, i, h, wsem)

        pltpu.sync_copy(pb_hbm.at[wid], pb_v)
        build(0, 0, 0)
        fire(0, buf0, sem0)
        build(1, 0, 1)
        fire(1, buf1, sem1)

        def body(t, carry):
            l0 = 2 * t
            l1 = l0 + 1
            l2 = jnp.minimum(l1 + 1, nblk - 1)
            q0 = i0 + l0
            q1 = q0 + 1

            build(2, l1, 0)
            fire(2, buf2, sem2)
            halfstep(buf0, sem0, outA, wsemA, q0, 0, t)
            build(3, l1, 1)
            fire(3, buf3, sem3)
            halfstep(buf1, sem1, outB, wsemB, q0, 1, t)
            build(0, l2, 0)
            fire(0, buf0, sem0)
            halfstep(buf2, sem2, outA, wsemA, q1, 0, t + 1)
            build(1, l2, 1)
            fire(1, buf1, sem1)
            halfstep(buf3, sem3, outB, wsemB, q1, 1, t + 1)
            return carry

        lax.fori_loop(0, nblk // 2, body, 0)

        drain_gather(buf0, sem0)
        drain_gather(buf1, sem1)
        drain_write(outA, wsemA)
        drain_write(outB, wsemB)

    return k


def kernel(x, indices, block_size, block_stride, block_offset):
    N, C, H, W = x.shape
    nB = indices.shape[0]
    wc = W // BW
    rows_total = N * C * H * wc

    n = indices[:, 0]
    ys = indices[:, 1] * block_stride[0] + block_offset[0]
    xs = indices[:, 2] * block_stride[1] + block_offset[1]

    h = ys[:, None] + jnp.arange(BH, dtype=jnp.int32)[None, :]
    tr = h >> 3
    hi = h & 7
    tc = (xs >> 7)[:, None]
    ck = ((xs & 127) >> 4)[:, None]
    prow16 = tr * ((W // 128) * 64) + hi * 8 + tc * 64 + ck
    pb = (prow16 + (n * C)[:, None] * (H * wc)).astype(jnp.int32)

    table = (x.reshape(N * C, H // 8, 8, W // 128, 128)
             .transpose(0, 1, 3, 2, 4)
             .reshape(rows_total, BW))
    NW = 32
    pb = pb.reshape(NW, nB // NW, BH)
    out = _make_sc_gather(nB, C, rows_total, H * wc)(table, pb)
    return out.reshape(nB, BH, BW, C)


# speedup vs baseline: 1.3586x; 1.0041x over previous
import functools

import jax
import jax.numpy as jnp
from jax import lax
from jax.experimental import pallas as pl
from jax.experimental.pallas import tpu as pltpu
from jax.experimental.pallas import tpu_sc as plsc

BH = 16
BW = 16
HH = BH // 2


def _make_sc_gather(nB, C, rows_total, plane_rows):
    info = plsc.get_sparse_core_info()
    NC, NS = info.num_cores, info.num_subcores
    NW = NC * NS
    nblk = nB // NW
    half_rows = HH * C

    mesh = plsc.VectorSubcoreMesh(core_axis_name="c", subcore_axis_name="s")

    @functools.partial(
        pl.kernel,
        mesh=mesh,
        compiler_params=pltpu.CompilerParams(
            needs_layout_passes=False, use_tc_tiling_on_sc=False
        ),
        out_type=jax.ShapeDtypeStruct((nB, 2, HH * BW, C), jnp.float32),
        scratch_types=[
            pltpu.VMEM((nblk, BH), jnp.int32),
            pltpu.VMEM((4, half_rows), jnp.int32),
            pltpu.VMEM((half_rows, BW), jnp.float32),
            pltpu.VMEM((half_rows, BW), jnp.float32),
            pltpu.VMEM((half_rows, BW), jnp.float32),
            pltpu.VMEM((half_rows, BW), jnp.float32),
            pltpu.VMEM((HH * BW, 129), jnp.float32),
            pltpu.VMEM((HH * BW, 129), jnp.float32),
            pltpu.SemaphoreType.DMA,
            pltpu.SemaphoreType.DMA,
            pltpu.SemaphoreType.DMA,
            pltpu.SemaphoreType.DMA,
            pltpu.SemaphoreType.DMA,
            pltpu.SemaphoreType.DMA,
        ],
    )
    def k(table_hbm, pb_hbm, out_hbm, pb_v, idx_v, buf0, buf1, buf2, buf3,
          outA, outB, sem0, sem1, sem2, sem3, wsemA, wsemB):
        wid = lax.axis_index("s") * NC + lax.axis_index("c")
        i0 = wid * nblk
        iota16 = lax.iota(jnp.int32, 16)
        zero16 = jnp.zeros((16,), jnp.int32)
        one16 = jnp.ones((16,), jnp.int32)
        coffs = [(iota16 + cg * 16) * plane_rows for cg in range(C // 16)]

        def build(slot, blk, h):
            rv = pb_v[blk, :]
            for a in range(HH):
                s = rv[h * HH + a]
                for cg in range(C // 16):
                    idx_v[slot, pl.ds(a * C + cg * 16, 16)] = coffs[cg] + s

        def fire(slot, buf_ref, sem):
            pltpu.async_copy(table_hbm.at[idx_v.at[slot]], buf_ref, sem)

        def drain_gather(buf_ref, sem):
            pltpu.make_async_copy(
                table_hbm.at[idx_v.at[0]], buf_ref, sem
            ).wait()

        def transpose(buf_ref, out_ref):
            def a_body(a, carry):
                rbase = a * C
                rowv = iota16 + a * BW

                @plsc.parallel_loop(0, C, unroll=16, carry=zero16)
                def tr(c, colv):
                    v = buf_ref[rbase + c, :]
                    plsc.store_scatter(out_ref, [rowv, colv], v)
                    return colv + one16

                return carry

            lax.fori_loop(0, HH, a_body, 0)

        def issue_write(out_ref, i, h, wsem):
            pltpu.async_copy(
                out_ref.at[:, pl.ds(0, C)], out_hbm.at[i, h], wsem
            )

        def drain_write(out_ref, wsem):
            pltpu.make_async_copy(
                out_ref.at[:, pl.ds(0, C)], out_hbm.at[0, 0], wsem
            ).wait()

        def halfstep(buf_ref, sem, out_ref, wsem, i, h, t):
            drain_gather(buf_ref, sem)

            @pl.when(t > 0)
            def _():
                drain_write(out_ref, wsem)

            transpose(buf_ref, out_ref)
            issue_write(out_ref, i, h, wsem)

        pltpu.sync_copy(pb_hbm.at[wid], pb_v)
        build(0, 0, 0)
        fire(0, buf0, sem0)
        build(1, 0, 1)
        fire(1, buf1, sem1)

        def body(t, carry):
            l0 = 2 * t
            l1 = l0 + 1
            l2 = jnp.minimum(l1 + 1, nblk - 1)
            q0 = i0 + l0
            q1 = q0 + 1

            build(2, l1, 0)
            fire(2, buf2, sem2)
            halfstep(buf0, sem0, outA, wsemA, q0, 0, t)
            build(3, l1, 1)
            fire(3, buf3, sem3)
            halfstep(buf1, sem1, outB, wsemB, q0, 1, t)
            build(0, l2, 0)
            fire(0, buf0, sem0)
            halfstep(buf2, sem2, outA, wsemA, q1, 0, t + 1)
            build(1, l2, 1)
            fire(1, buf1, sem1)
            halfstep(buf3, sem3, outB, wsemB, q1, 1, t + 1)
            return carry

        lax.fori_loop(0, nblk // 2, body, 0)

        drain_gather(buf0, sem0)
        drain_gather(buf1, sem1)
        drain_write(outA, wsemA)
        drain_write(outB, wsemB)

    return k


def kernel(x, indices, block_size, block_stride, block_offset):
    N, C, H, W = x.shape
    nB = indices.shape[0]
    wc = W // BW
    rows_total = N * C * H * wc

    n = indices[:, 0]
    ys = indices[:, 1] * block_stride[0] + block_offset[0]
    xs = indices[:, 2] * block_stride[1] + block_offset[1]

    h = ys[:, None] + jnp.arange(BH, dtype=jnp.int32)[None, :]
    tr = h >> 3
    hi = h & 7
    tc = (xs >> 7)[:, None]
    ck = ((xs & 127) >> 4)[:, None]
    prow16 = tr * ((W // 128) * 64) + hi * 8 + tc * 64 + ck
    pb = (prow16 + (n * C)[:, None] * (H * wc)).astype(jnp.int32)

    table = (x.reshape(N * C, H // 8, 8, W // 128, 128)
             .transpose(0, 1, 3, 2, 4)
             .reshape(rows_total, BW))
    info = plsc.get_sparse_core_info()
    NW = info.num_cores * info.num_subcores
    pb = pb.reshape(NW, nB // NW, BH)
    out = _make_sc_gather(nB, C, rows_total, H * wc)(table, pb)
    return out.reshape(nB, BH, BW, C)
